# baseline (speedup 1.0000x reference)
import functools
import jax
import jax.numpy as jnp
from jax import lax
from jax.experimental import pallas as pl
from jax.experimental.pallas import tpu as pltpu
from jax.experimental.pallas import tpu_sc as plsc

_H = 128
_K = 16
_L = 4


def _gn(x, gamma, beta, eps=1e-5):
    mu = jnp.mean(x, axis=-1, keepdims=True)
    xc = x - mu
    var = jnp.mean(xc * xc, axis=-1, keepdims=True)
    return xc * (lax.rsqrt(var + eps) * gamma) + beta


def _body_a(x_ref, fr_ref, frt_ref, Win_ref, bin_ref, Win2_ref, gin_ref,
            bin2_ref, h_ref, d2_ref):
    f32 = jnp.float32
    x = x_ref[0]
    h = jnp.dot(x, Win_ref[...], preferred_element_type=f32) + bin_ref[...]
    h = jnp.maximum(h, 0.0)
    h = jnp.dot(h, Win2_ref[...], preferred_element_type=f32)
    h = _gn(h, gin_ref[...], bin2_ref[...])
    h_ref[0] = jnp.maximum(h, 0.0)

    fr = fr_ref[0]
    frt = frt_ref[0]
    d2 = None
    for c in range(3):
        d = fr[:, c:c + 1] - frt[c:c + 1, :]
        d2 = d * d if d2 is None else d2 + d * d
    d2_ref[...] = d2


def _sc_topk_thresh(d2_2d, n_rows, n_cand):
    NC, NS = 2, 16
    NW = NC * NS
    rows_per_w = n_rows // NW
    BLK = 16
    n_blk = rows_per_w // BLK
    ILV = 8
    mesh = plsc.VectorSubcoreMesh(core_axis_name="c", subcore_axis_name="s",
                                  num_cores=NC, num_subcores=NS)

    @functools.partial(
        pl.kernel, mesh=mesh,
        compiler_params=pltpu.CompilerParams(needs_layout_passes=False),
        out_type=jax.ShapeDtypeStruct((n_rows,), jnp.float32),
        scratch_types=[
            pltpu.VMEM((BLK, n_cand), jnp.float32),
            pltpu.VMEM((BLK, n_cand), jnp.float32),
            pltpu.VMEM((rows_per_w,), jnp.float32),
            pltpu.SemaphoreType.DMA,
            pltpu.SemaphoreType.DMA,
        ],
    )
    def k(d2_hbm, out_hbm, rows_a, rows_b, thr_v, sem_a, sem_b):
        wid = lax.axis_index("s") * NC + lax.axis_index("c")
        base_row = wid * rows_per_w
        big = jnp.float32(3.4e38)
        lane = lax.iota(jnp.int32, 16)
        bufs = (rows_a, rows_b)
        sems = (sem_a, sem_b)

        def start(blk, side):
            pltpu.make_async_copy(
                d2_hbm.at[pl.ds(base_row + blk * BLK, BLK)],
                bufs[side], sems[side]).start()

        def wait(blk, side):
            pltpu.make_async_copy(
                d2_hbm.at[pl.ds(base_row + blk * BLK, BLK)],
                bufs[side], sems[side]).wait()

        def process(blk, side):
            rows_v = bufs[side]
            thr_vec = jnp.zeros((16,), jnp.float32)
            for rg in range(0, BLK, ILV):
                def chunk(j, bests):
                    new = []
                    for q in range(ILV):
                        ck = rows_v[rg + q, pl.ds(j * 16, 16)]
                        ck = lax.sort(ck)
                        lo = jnp.minimum(bests[q], lax.rev(ck, (0,)))
                        new.append(lax.sort(lo))
                    return tuple(new)

                init = tuple(jnp.full((16,), big, jnp.float32)
                             for _ in range(ILV))
                bests = lax.fori_loop(0, n_cand // 16, chunk, init)
                for q in range(ILV):
                    thr_vec = jnp.where(lane == rg + q, jnp.max(bests[q]),
                                        thr_vec)
            thr_v[pl.ds(blk * BLK, 16)] = thr_vec

        start(0, 0)
        start(1, 1)

        def pair_body(p, carry):
            blk0 = p * 2
            wait(blk0, 0)
            process(blk0, 0)

            @pl.when(blk0 + 2 < n_blk)
            def _():
                start(blk0 + 2, 0)

            wait(blk0 + 1, 1)
            process(blk0 + 1, 1)

            @pl.when(blk0 + 3 < n_blk)
            def _():
                start(blk0 + 3, 1)
            return carry

        lax.fori_loop(0, n_blk // 2, pair_body, jnp.int32(0))
        pltpu.sync_copy(thr_v, out_hbm.at[pl.ds(base_row, rows_per_w)])

    return k(d2_2d)


def _body_b(h_ref, d2_ref, thr_ref, Wf_ref, We_ref, ng_ref, nb_ref,
            cW_ref, cg_ref, cb_ref, out_ref):
    f32 = jnp.float32
    h = h_ref[0]
    res = h
    d2 = d2_ref[...]
    thr = thr_ref[0]
    adj = (d2 <= thr).astype(f32)
    for i in range(_L):
        x_node = jnp.dot(h, Wf_ref[i], preferred_element_type=f32)
        x_edge = jnp.dot(h, We_ref[i], preferred_element_type=f32)
        tmp = jnp.dot(adj, x_edge, preferred_element_type=f32)
        h = _gn(x_node + tmp, ng_ref[i:i + 1], nb_ref[i:i + 1])
        h = jnp.maximum(h, 0.0)
        h = _gn(jnp.dot(h, cW_ref[i], preferred_element_type=f32),
                cg_ref[i:i + 1], cb_ref[i:i + 1])
        h = jnp.maximum(h + res, 0.0)
        res = h
    out_ref[0] = h


def kernel(x, frames, W_in, b_in, W_in2, g_in, b_in2, W_fuse, W_edge,
           norm_g, norm_b, ctr2_W, ctr2_g, ctr2_b):
    B, S, _ = x.shape
    frames_t = jnp.transpose(frames, (0, 2, 1))

    def full(a):
        return pl.BlockSpec(a.shape, lambda b: (0,) * a.ndim)

    b_in2d = b_in.reshape(1, _H)
    g_in2d = g_in.reshape(1, _H)
    b_in22d = b_in2.reshape(1, _H)

    G = 8
    gB = B // G

    def call_a(xg, frg, frtg):
        return pl.pallas_call(
            _body_a,
            grid=(gB,),
            in_specs=[
                pl.BlockSpec((1, S, 20), lambda b: (b, 0, 0)),
                pl.BlockSpec((1, S, 3), lambda b: (b, 0, 0)),
                pl.BlockSpec((1, 3, S), lambda b: (b, 0, 0)),
                full(W_in), full(b_in2d), full(W_in2), full(g_in2d),
                full(b_in22d),
            ],
            out_specs=[
                pl.BlockSpec((1, S, _H), lambda b: (b, 0, 0)),
                pl.BlockSpec((S, S), lambda b: (b, 0)),
            ],
            out_shape=[
                jax.ShapeDtypeStruct((gB, S, _H), jnp.float32),
                jax.ShapeDtypeStruct((gB * S, S), jnp.float32),
            ],
            compiler_params=pltpu.CompilerParams(
                dimension_semantics=("parallel",)),
        )(xg, frg, frtg, W_in, b_in2d, W_in2, g_in2d, b_in22d)

    def call_b(h0g, d2g, thr3g):
        return pl.pallas_call(
            _body_b,
            grid=(gB,),
            in_specs=[
                pl.BlockSpec((1, S, _H), lambda b: (b, 0, 0)),
                pl.BlockSpec((S, S), lambda b: (b, 0)),
                pl.BlockSpec((1, S, 1), lambda b: (b, 0, 0)),
                full(W_fuse), full(W_edge), full(norm_g), full(norm_b),
                full(ctr2_W), full(ctr2_g), full(ctr2_b),
            ],
            out_specs=pl.BlockSpec((1, S, _H), lambda b: (b, 0, 0)),
            out_shape=jax.ShapeDtypeStruct((gB, S, _H), jnp.float32),
            compiler_params=pltpu.CompilerParams(
                dimension_semantics=("parallel",)),
        )(h0g, d2g, thr3g, W_fuse, W_edge, norm_g, norm_b, ctr2_W,
          ctr2_g, ctr2_b)

    h0s, d2s, thrs = [], [], []
    for g in range(G):
        sl = slice(g * gB, (g + 1) * gB)
        h0g, d2g = call_a(x[sl], frames[sl], frames_t[sl])
        h0s.append(h0g)
        d2s.append(d2g)
    for g in range(G):
        thrs.append(_sc_topk_thresh(d2s[g], gB * S, S))
    outs = []
    for g in range(G):
        outs.append(call_b(h0s[g], d2s[g], thrs[g].reshape(gB, S, 1)))
    return jnp.concatenate(outs, axis=0)


# speedup vs baseline: 1.2277x; 1.2277x over previous
import functools
import jax
import jax.numpy as jnp
from jax import lax
from jax.experimental import pallas as pl
from jax.experimental.pallas import tpu as pltpu
from jax.experimental.pallas import tpu_sc as plsc

_H = 128
_K = 16
_L = 4


def _gn(x, gamma, beta, eps=1e-5):
    mu = jnp.mean(x, axis=-1, keepdims=True)
    xc = x - mu
    var = jnp.mean(xc * xc, axis=-1, keepdims=True)
    return xc * (lax.rsqrt(var + eps) * gamma) + beta


def _body_a(x_ref, fr_ref, frt_ref, Win_ref, bin_ref, Win2_ref, gin_ref,
            bin2_ref, h_ref, d2_ref):
    f32 = jnp.float32
    x = x_ref[0]
    h = jnp.dot(x, Win_ref[...], preferred_element_type=f32) + bin_ref[...]
    h = jnp.maximum(h, 0.0)
    h = jnp.dot(h, Win2_ref[...], preferred_element_type=f32)
    h = _gn(h, gin_ref[...], bin2_ref[...])
    h_ref[0] = jnp.maximum(h, 0.0)

    fr = fr_ref[0]
    frt = frt_ref[0]
    d2 = None
    for c in range(3):
        d = fr[:, c:c + 1] - frt[c:c + 1, :]
        d2 = d * d if d2 is None else d2 + d * d
    d2_ref[...] = d2


def _sc_topk_thresh(d2_2d, row0, n_rows, n_cand):
    NC, NS = 2, 16
    NW = NC * NS
    rows_per_w = n_rows // NW
    BLK = 16
    n_blk = rows_per_w // BLK
    ILV = 8
    mesh = plsc.VectorSubcoreMesh(core_axis_name="c", subcore_axis_name="s",
                                  num_cores=NC, num_subcores=NS)

    @functools.partial(
        pl.kernel, mesh=mesh,
        compiler_params=pltpu.CompilerParams(needs_layout_passes=False),
        out_type=jax.ShapeDtypeStruct((n_rows,), jnp.float32),
        scratch_types=[
            pltpu.VMEM((BLK, n_cand), jnp.float32),
            pltpu.VMEM((BLK, n_cand), jnp.float32),
            pltpu.VMEM((rows_per_w,), jnp.float32),
            pltpu.SemaphoreType.DMA,
            pltpu.SemaphoreType.DMA,
        ],
    )
    def k(d2_hbm, out_hbm, rows_a, rows_b, thr_v, sem_a, sem_b):
        wid = lax.axis_index("s") * NC + lax.axis_index("c")
        base_out = wid * rows_per_w
        base_row = row0 + base_out
        big = jnp.float32(3.4e38)
        lane = lax.iota(jnp.int32, 16)
        bufs = (rows_a, rows_b)
        sems = (sem_a, sem_b)

        def start(blk, side):
            pltpu.make_async_copy(
                d2_hbm.at[pl.ds(base_row + blk * BLK, BLK)],
                bufs[side], sems[side]).start()

        def wait(blk, side):
            pltpu.make_async_copy(
                d2_hbm.at[pl.ds(base_row + blk * BLK, BLK)],
                bufs[side], sems[side]).wait()

        def process(blk, side):
            rows_v = bufs[side]
            thr_vec = jnp.zeros((16,), jnp.float32)
            for rg in range(0, BLK, ILV):
                def chunk(j, bests):
                    new = []
                    for q in range(ILV):
                        ck = rows_v[rg + q, pl.ds(j * 16, 16)]
                        ck = lax.sort(ck)
                        lo = jnp.minimum(bests[q], lax.rev(ck, (0,)))
                        new.append(lax.sort(lo))
                    return tuple(new)

                init = tuple(jnp.full((16,), big, jnp.float32)
                             for _ in range(ILV))
                bests = lax.fori_loop(0, n_cand // 16, chunk, init)
                for q in range(ILV):
                    thr_vec = jnp.where(lane == rg + q, jnp.max(bests[q]),
                                        thr_vec)
            thr_v[pl.ds(blk * BLK, 16)] = thr_vec

        start(0, 0)
        start(1, 1)

        def pair_body(p, carry):
            blk0 = p * 2
            wait(blk0, 0)
            process(blk0, 0)

            @pl.when(blk0 + 2 < n_blk)
            def _():
                start(blk0 + 2, 0)

            wait(blk0 + 1, 1)
            process(blk0 + 1, 1)

            @pl.when(blk0 + 3 < n_blk)
            def _():
                start(blk0 + 3, 1)
            return carry

        lax.fori_loop(0, n_blk // 2, pair_body, jnp.int32(0))
        pltpu.sync_copy(thr_v, out_hbm.at[pl.ds(base_out, rows_per_w)])

    return k(d2_2d)


def _body_b(h_ref, d2_ref, thr_ref, Wf_ref, We_ref, ng_ref, nb_ref,
            cW_ref, cg_ref, cb_ref, out_ref):
    f32 = jnp.float32
    h = h_ref[0]
    res = h
    d2 = d2_ref[...]
    thr = thr_ref[0]
    adj = (d2 <= thr).astype(f32)
    for i in range(_L):
        x_node = jnp.dot(h, Wf_ref[i], preferred_element_type=f32)
        x_edge = jnp.dot(h, We_ref[i], preferred_element_type=f32)
        tmp = jnp.dot(adj, x_edge, preferred_element_type=f32)
        h = _gn(x_node + tmp, ng_ref[i:i + 1], nb_ref[i:i + 1])
        h = jnp.maximum(h, 0.0)
        h = _gn(jnp.dot(h, cW_ref[i], preferred_element_type=f32),
                cg_ref[i:i + 1], cb_ref[i:i + 1])
        h = jnp.maximum(h + res, 0.0)
        res = h
    out_ref[0] = h


def kernel(x, frames, W_in, b_in, W_in2, g_in, b_in2, W_fuse, W_edge,
           norm_g, norm_b, ctr2_W, ctr2_g, ctr2_b):
    B, S, _ = x.shape
    frames_t = jnp.transpose(frames, (0, 2, 1))

    def full(a):
        return pl.BlockSpec(a.shape, lambda b: (0,) * a.ndim)

    b_in2d = b_in.reshape(1, _H)
    g_in2d = g_in.reshape(1, _H)
    b_in22d = b_in2.reshape(1, _H)

    G = 4
    gB = B // G

    h0, d2 = pl.pallas_call(
        _body_a,
        grid=(B,),
        in_specs=[
            pl.BlockSpec((1, S, 20), lambda b: (b, 0, 0)),
            pl.BlockSpec((1, S, 3), lambda b: (b, 0, 0)),
            pl.BlockSpec((1, 3, S), lambda b: (b, 0, 0)),
            full(W_in), full(b_in2d), full(W_in2), full(g_in2d),
            full(b_in22d),
        ],
        out_specs=[
            pl.BlockSpec((1, S, _H), lambda b: (b, 0, 0)),
            pl.BlockSpec((S, S), lambda b: (b, 0)),
        ],
        out_shape=[
            jax.ShapeDtypeStruct((B, S, _H), jnp.float32),
            jax.ShapeDtypeStruct((B * S, S), jnp.float32),
        ],
        compiler_params=pltpu.CompilerParams(
            dimension_semantics=("parallel",)),
    )(x, frames, frames_t, W_in, b_in2d, W_in2, g_in2d, b_in22d)

    def call_b(g, thr3g):
        return pl.pallas_call(
            _body_b,
            grid=(gB,),
            in_specs=[
                pl.BlockSpec((1, S, _H), lambda b: (g * gB + b, 0, 0)),
                pl.BlockSpec((S, S), lambda b: (g * gB + b, 0)),
                pl.BlockSpec((1, S, 1), lambda b: (b, 0, 0)),
                full(W_fuse), full(W_edge), full(norm_g), full(norm_b),
                full(ctr2_W), full(ctr2_g), full(ctr2_b),
            ],
            out_specs=pl.BlockSpec((1, S, _H), lambda b: (b, 0, 0)),
            out_shape=jax.ShapeDtypeStruct((gB, S, _H), jnp.float32),
            compiler_params=pltpu.CompilerParams(
                dimension_semantics=("parallel",)),
        )(h0, d2, thr3g, W_fuse, W_edge, norm_g, norm_b, ctr2_W,
          ctr2_g, ctr2_b)

    thrs = [_sc_topk_thresh(d2, g * gB * S, gB * S, S) for g in range(G)]
    outs = [call_b(g, thrs[g].reshape(gB, S, 1)) for g in range(G)]
    return jnp.concatenate(outs, axis=0)


# speedup vs baseline: 1.2486x; 1.0170x over previous
import functools
import jax
import jax.numpy as jnp
from jax import lax
from jax.experimental import pallas as pl
from jax.experimental.pallas import tpu as pltpu
from jax.experimental.pallas import tpu_sc as plsc

_H = 128
_K = 16
_L = 4


def _gn(x, gamma, beta, eps=1e-5):
    mu = jnp.mean(x, axis=-1, keepdims=True)
    xc = x - mu
    var = jnp.mean(xc * xc, axis=-1, keepdims=True)
    return xc * (lax.rsqrt(var + eps) * gamma) + beta


def _body_a(x_ref, fr_ref, frt_ref, Win_ref, bin_ref, Win2_ref, gin_ref,
            bin2_ref, h_ref, d2_ref):
    f32 = jnp.float32
    x = x_ref[0]
    h = jnp.dot(x, Win_ref[...], preferred_element_type=f32) + bin_ref[...]
    h = jnp.maximum(h, 0.0)
    h = jnp.dot(h, Win2_ref[...], preferred_element_type=f32)
    h = _gn(h, gin_ref[...], bin2_ref[...])
    h_ref[0] = jnp.maximum(h, 0.0)

    fr = fr_ref[0]
    frt = frt_ref[0]
    d2 = None
    for c in range(3):
        d = fr[:, c:c + 1] - frt[c:c + 1, :]
        d2 = d * d if d2 is None else d2 + d * d
    d2_ref[...] = d2


def _sc_topk_thresh(d2_2d, row0, n_rows, n_cand):
    NC, NS = 2, 16
    NW = NC * NS
    rows_per_w = n_rows // NW
    BLK = 16
    n_blk = rows_per_w // BLK
    ILV = 8
    mesh = plsc.VectorSubcoreMesh(core_axis_name="c", subcore_axis_name="s",
                                  num_cores=NC, num_subcores=NS)

    @functools.partial(
        pl.kernel, mesh=mesh,
        compiler_params=pltpu.CompilerParams(needs_layout_passes=False),
        out_type=jax.ShapeDtypeStruct((n_rows,), jnp.float32),
        scratch_types=[
            pltpu.VMEM((BLK, n_cand), jnp.float32),
            pltpu.VMEM((BLK, n_cand), jnp.float32),
            pltpu.VMEM((rows_per_w,), jnp.float32),
            pltpu.SemaphoreType.DMA,
            pltpu.SemaphoreType.DMA,
        ],
    )
    def k(d2_hbm, out_hbm, rows_a, rows_b, thr_v, sem_a, sem_b):
        wid = lax.axis_index("s") * NC + lax.axis_index("c")
        base_out = wid * rows_per_w
        base_row = row0 + base_out
        big = jnp.float32(3.4e38)
        lane = lax.iota(jnp.int32, 16)
        bufs = (rows_a, rows_b)
        sems = (sem_a, sem_b)

        def start(blk, side):
            pltpu.make_async_copy(
                d2_hbm.at[pl.ds(base_row + blk * BLK, BLK)],
                bufs[side], sems[side]).start()

        def wait(blk, side):
            pltpu.make_async_copy(
                d2_hbm.at[pl.ds(base_row + blk * BLK, BLK)],
                bufs[side], sems[side]).wait()

        def process(blk, side):
            rows_v = bufs[side]
            thr_vec = jnp.zeros((16,), jnp.float32)
            for rg in range(0, BLK, ILV):
                def chunk(j, bests):
                    new = []
                    for q in range(ILV):
                        ck = rows_v[rg + q, pl.ds(j * 16, 16)]
                        ck = lax.sort(ck)
                        lo = jnp.minimum(bests[q], lax.rev(ck, (0,)))
                        new.append(lax.sort(lo))
                    return tuple(new)

                init = tuple(jnp.full((16,), big, jnp.float32)
                             for _ in range(ILV))
                bests = lax.fori_loop(0, n_cand // 16, chunk, init)
                for q in range(ILV):
                    thr_vec = jnp.where(lane == rg + q, jnp.max(bests[q]),
                                        thr_vec)
            thr_v[pl.ds(blk * BLK, 16)] = thr_vec

        start(0, 0)
        start(1, 1)

        def pair_body(p, carry):
            blk0 = p * 2
            wait(blk0, 0)
            process(blk0, 0)

            @pl.when(blk0 + 2 < n_blk)
            def _():
                start(blk0 + 2, 0)

            wait(blk0 + 1, 1)
            process(blk0 + 1, 1)

            @pl.when(blk0 + 3 < n_blk)
            def _():
                start(blk0 + 3, 1)
            return carry

        lax.fori_loop(0, n_blk // 2, pair_body, jnp.int32(0))
        pltpu.sync_copy(thr_v, out_hbm.at[pl.ds(base_out, rows_per_w)])

    return k(d2_2d)


def _body_b(h_ref, d2_ref, thr_ref, Wf_ref, We_ref, ng_ref, nb_ref,
            cW_ref, cg_ref, cb_ref, out_ref):
    f32 = jnp.float32
    h = h_ref[0]
    res = h
    d2 = d2_ref[...]
    thr = thr_ref[0]
    adj = (d2 <= thr).astype(f32)
    for i in range(_L):
        x_node = jnp.dot(h, Wf_ref[i], preferred_element_type=f32)
        x_edge = jnp.dot(h, We_ref[i], preferred_element_type=f32)
        tmp = jnp.dot(adj, x_edge, preferred_element_type=f32)
        h = _gn(x_node + tmp, ng_ref[i:i + 1], nb_ref[i:i + 1])
        h = jnp.maximum(h, 0.0)
        h = _gn(jnp.dot(h, cW_ref[i], preferred_element_type=f32),
                cg_ref[i:i + 1], cb_ref[i:i + 1])
        h = jnp.maximum(h + res, 0.0)
        res = h
    out_ref[0] = h


def kernel(x, frames, W_in, b_in, W_in2, g_in, b_in2, W_fuse, W_edge,
           norm_g, norm_b, ctr2_W, ctr2_g, ctr2_b):
    B, S, _ = x.shape
    frames_t = jnp.transpose(frames, (0, 2, 1))

    def full(a):
        return pl.BlockSpec(a.shape, lambda b: (0,) * a.ndim)

    b_in2d = b_in.reshape(1, _H)
    g_in2d = g_in.reshape(1, _H)
    b_in22d = b_in2.reshape(1, _H)

    G = 2
    gB = B // G

    h0, d2 = pl.pallas_call(
        _body_a,
        grid=(B,),
        in_specs=[
            pl.BlockSpec((1, S, 20), lambda b: (b, 0, 0)),
            pl.BlockSpec((1, S, 3), lambda b: (b, 0, 0)),
            pl.BlockSpec((1, 3, S), lambda b: (b, 0, 0)),
            full(W_in), full(b_in2d), full(W_in2), full(g_in2d),
            full(b_in22d),
        ],
        out_specs=[
            pl.BlockSpec((1, S, _H), lambda b: (b, 0, 0)),
            pl.BlockSpec((S, S), lambda b: (b, 0)),
        ],
        out_shape=[
            jax.ShapeDtypeStruct((B, S, _H), jnp.float32),
            jax.ShapeDtypeStruct((B * S, S), jnp.float32),
        ],
        compiler_params=pltpu.CompilerParams(
            dimension_semantics=("parallel",)),
    )(x, frames, frames_t, W_in, b_in2d, W_in2, g_in2d, b_in22d)

    def call_b(g, thr3g):
        return pl.pallas_call(
            _body_b,
            grid=(gB,),
            in_specs=[
                pl.BlockSpec((1, S, _H), lambda b: (g * gB + b, 0, 0)),
                pl.BlockSpec((S, S), lambda b: (g * gB + b, 0)),
                pl.BlockSpec((1, S, 1), lambda b: (b, 0, 0)),
                full(W_fuse), full(W_edge), full(norm_g), full(norm_b),
                full(ctr2_W), full(ctr2_g), full(ctr2_b),
            ],
            out_specs=pl.BlockSpec((1, S, _H), lambda b: (b, 0, 0)),
            out_shape=jax.ShapeDtypeStruct((gB, S, _H), jnp.float32),
            compiler_params=pltpu.CompilerParams(
                dimension_semantics=("parallel",)),
        )(h0, d2, thr3g, W_fuse, W_edge, norm_g, norm_b, ctr2_W,
          ctr2_g, ctr2_b)

    thrs = [_sc_topk_thresh(d2, g * gB * S, gB * S, S) for g in range(G)]
    outs = [call_b(g, thrs[g].reshape(gB, S, 1)) for g in range(G)]
    return jnp.concatenate(outs, axis=0)
